# Initial kernel scaffold; baseline (speedup 1.0000x reference)
import functools

import jax
import jax.numpy as jnp
from jax import lax
from jax.experimental import pallas as pl
from jax.experimental.pallas import tpu as pltpu
from jax.experimental.pallas import tpu_sc as plsc

_NUM_CORES = 2
_NUM_SUBCORES = 16
_NUM_TILES = _NUM_CORES * _NUM_SUBCORES
_LANES = 16


def _build(n_ids: int, chunk: int):
    per_tile = n_ids // _NUM_TILES
    assert per_tile % chunk == 0
    n_chunks = per_tile // chunk

    mesh = plsc.VectorSubcoreMesh(core_axis_name="c", subcore_axis_name="s")

    @functools.partial(
        pl.kernel,
        out_type=jax.ShapeDtypeStruct((2 * n_ids,), jnp.float32),
        mesh=mesh,
        scratch_types=[
            pltpu.VMEM((chunk,), jnp.int32),
            pltpu.VMEM((2 * chunk,), jnp.float32),
            pltpu.VMEM((16,), jnp.float32),
            pltpu.VMEM((48,), jnp.float32),
        ],
    )
    def fused_lookup(ids_hbm, w_hbm, out_hbm, ids_v, out_v, t16, wbuf):
        pltpu.sync_copy(w_hbm, wbuf)

        jj = lax.iota(jnp.int32, _LANES)
        v = jj >> 1
        c = jj & 1
        acc = plsc.load_gather(wbuf, [c + 40])
        for d in range(4):
            e = plsc.load_gather(wbuf, [v * 4 + d])
            w = plsc.load_gather(wbuf, [c * 4 + (32 + d)])
            acc = acc + e * w
        t16[...] = acc

        wid = lax.axis_index("s") * _NUM_CORES + lax.axis_index("c")
        base = wid * per_tile
        jj2 = jj * 2

        def chunk_body(k, carry):
            off = base + k * chunk
            pltpu.sync_copy(ids_hbm.at[pl.ds(off, chunk)], ids_v)

            def vec_body(i, carry2):
                ids16 = ids_v[pl.ds(i * _LANES, _LANES)]
                key = ids16 * 2
                e0 = plsc.load_gather(t16, [key])
                e1 = plsc.load_gather(t16, [key + 1])
                pos = i * (2 * _LANES) + jj2
                plsc.store_scatter(out_v, [pos], e0)
                plsc.store_scatter(out_v, [pos + 1], e1)
                return carry2

            lax.fori_loop(0, chunk // _LANES, vec_body, 0)
            pltpu.sync_copy(out_v, out_hbm.at[pl.ds(2 * off, 2 * chunk)])
            return carry

        lax.fori_loop(0, n_chunks, chunk_body, 0)

    return fused_lookup


def kernel(input_ids, embed_weight, lm_head_weight, lm_head_bias):
    b, l = input_ids.shape
    n_ids = b * l
    ids_flat = input_ids.reshape(-1).astype(jnp.int32)
    wpacked = jnp.concatenate(
        [
            embed_weight.reshape(-1).astype(jnp.float32),
            lm_head_weight.reshape(-1).astype(jnp.float32),
            lm_head_bias.astype(jnp.float32),
            jnp.zeros((6,), jnp.float32),
        ]
    )
    out_flat = _build(n_ids, chunk=10240)(ids_flat, wpacked)
    return out_flat.reshape(b, l, 2)


# speedup vs baseline: 5.4354x; 5.4354x over previous
import functools

import jax
import jax.numpy as jnp
from jax import lax
from jax.experimental import pallas as pl
from jax.experimental.pallas import tpu as pltpu
from jax.experimental.pallas import tpu_sc as plsc

_NUM_CORES = 2
_NUM_SUBCORES = 16
_NUM_TILES = _NUM_CORES * _NUM_SUBCORES
_LANES = 16


def _build(n_ids: int, chunk: int):
    per_tile = n_ids // _NUM_TILES
    assert per_tile % chunk == 0
    n_chunks = per_tile // chunk

    mesh = plsc.VectorSubcoreMesh(core_axis_name="c", subcore_axis_name="s")

    @functools.partial(
        pl.kernel,
        out_type=jax.ShapeDtypeStruct((2 * n_ids,), jnp.float32),
        mesh=mesh,
        scratch_types=[
            pltpu.VMEM((chunk,), jnp.int32),
            pltpu.VMEM((2 * chunk,), jnp.float32),
            pltpu.VMEM((16,), jnp.float32),
            pltpu.VMEM((48,), jnp.float32),
        ],
        compiler_params=pltpu.CompilerParams(needs_layout_passes=False),
    )
    def fused_lookup(ids_hbm, w_hbm, out_hbm, ids_v, out_v, t16, wbuf):
        pltpu.sync_copy(w_hbm, wbuf)

        jj = lax.iota(jnp.int32, _LANES)
        v = jj >> 1
        c = jj & 1
        acc = plsc.load_gather(wbuf, [c + 40])
        for d in range(4):
            e = plsc.load_gather(wbuf, [v * 4 + d])
            w = plsc.load_gather(wbuf, [c * 4 + (32 + d)])
            acc = acc + e * w
        t16[...] = acc

        wid = lax.axis_index("s") * _NUM_CORES + lax.axis_index("c")
        base = wid * per_tile
        jj2 = jj * 2

        def chunk_body(k, carry):
            off = base + k * chunk
            pltpu.sync_copy(ids_hbm.at[pl.ds(off, chunk)], ids_v)

            def vec_body(i, carry2):
                ids16 = ids_v[pl.ds(i * _LANES, _LANES)]
                key = ids16 * 2
                e0 = plsc.load_gather(t16, [key])
                e1 = plsc.load_gather(t16, [key + 1])
                pos = i * (2 * _LANES) + jj2
                plsc.store_scatter(out_v, [pos], e0)
                plsc.store_scatter(out_v, [pos + 1], e1)
                return carry2

            lax.fori_loop(0, chunk // _LANES, vec_body, 0)
            pltpu.sync_copy(out_v, out_hbm.at[pl.ds(2 * off, 2 * chunk)])
            return carry

        lax.fori_loop(0, n_chunks, chunk_body, 0)

    return fused_lookup


def kernel(input_ids, embed_weight, lm_head_weight, lm_head_bias):
    b, l = input_ids.shape
    n_ids = b * l
    ids_flat = input_ids.reshape(-1).astype(jnp.int32)
    wpacked = jnp.concatenate(
        [
            embed_weight.reshape(-1).astype(jnp.float32),
            lm_head_weight.reshape(-1).astype(jnp.float32),
            lm_head_bias.astype(jnp.float32),
            jnp.zeros((6,), jnp.float32),
        ]
    )
    out_flat = _build(n_ids, chunk=10240)(ids_flat, wpacked)
    return out_flat.reshape(b, l, 2)


# speedup vs baseline: 5.5652x; 1.0239x over previous
import functools

import jax
import jax.numpy as jnp
from jax import lax
from jax.experimental import pallas as pl
from jax.experimental.pallas import tpu as pltpu
from jax.experimental.pallas import tpu_sc as plsc

_NUM_CORES = 2
_NUM_SUBCORES = 16
_NUM_TILES = _NUM_CORES * _NUM_SUBCORES
_LANES = 16


def _build(n_ids: int, chunk: int):
    per_tile = n_ids // _NUM_TILES
    assert per_tile % chunk == 0
    n_chunks = per_tile // chunk

    mesh = plsc.VectorSubcoreMesh(core_axis_name="c", subcore_axis_name="s")

    @functools.partial(
        pl.kernel,
        out_type=jax.ShapeDtypeStruct((2 * n_ids,), jnp.float32),
        mesh=mesh,
        scratch_types=[
            pltpu.VMEM((chunk,), jnp.int32),
            pltpu.VMEM((2 * chunk,), jnp.float32),
            pltpu.VMEM((16,), jnp.float32),
            pltpu.VMEM((48,), jnp.float32),
        ],
        compiler_params=pltpu.CompilerParams(needs_layout_passes=False),
    )
    def fused_lookup(ids_hbm, w_hbm, out_hbm, ids_v, out_v, t16, wbuf):
        pltpu.sync_copy(w_hbm, wbuf)

        jj = lax.iota(jnp.int32, _LANES)
        v = jj >> 1
        c = jj & 1
        acc = plsc.load_gather(wbuf, [c + 40])
        for d in range(4):
            e = plsc.load_gather(wbuf, [v * 4 + d])
            w = plsc.load_gather(wbuf, [c * 4 + (32 + d)])
            acc = acc + e * w
        t16[...] = acc

        wid = lax.axis_index("s") * _NUM_CORES + lax.axis_index("c")
        base = wid * per_tile
        jj2 = jj * 2

        def chunk_body(k, carry):
            off = base + k * chunk
            pltpu.sync_copy(ids_hbm.at[pl.ds(off, chunk)], ids_v)

            @plsc.parallel_loop(0, chunk, step=_LANES, unroll=8)
            def vec_body(i):
                ids16 = ids_v[pl.ds(i, _LANES)]
                key = ids16 * 2
                e0 = plsc.load_gather(t16, [key])
                e1 = plsc.load_gather(t16, [key + 1])
                pos = i * 2 + jj2
                plsc.store_scatter(out_v, [pos], e0)
                plsc.store_scatter(out_v, [pos + 1], e1)

            pltpu.sync_copy(out_v, out_hbm.at[pl.ds(2 * off, 2 * chunk)])
            return carry

        lax.fori_loop(0, n_chunks, chunk_body, 0)

    return fused_lookup


def kernel(input_ids, embed_weight, lm_head_weight, lm_head_bias):
    b, l = input_ids.shape
    n_ids = b * l
    ids_flat = input_ids.reshape(-1).astype(jnp.int32)
    wpacked = jnp.concatenate(
        [
            embed_weight.reshape(-1).astype(jnp.float32),
            lm_head_weight.reshape(-1).astype(jnp.float32),
            lm_head_bias.astype(jnp.float32),
            jnp.zeros((6,), jnp.float32),
        ]
    )
    out_flat = _build(n_ids, chunk=10240)(ids_flat, wpacked)
    return out_flat.reshape(b, l, 2)


# speedup vs baseline: 135.5635x; 24.3592x over previous
import functools

import jax
import jax.numpy as jnp
from jax import lax
from jax.experimental import pallas as pl
from jax.experimental.pallas import tpu as pltpu
from jax.experimental.pallas import tpu_sc as plsc

_NUM_CORES = 2
_NUM_SUBCORES = 16
_NUM_TILES = _NUM_CORES * _NUM_SUBCORES
_LANES = 16

_LBAND = 8


def _build(n_l: int, n_b: int):
    b_per_tile = n_b // _NUM_TILES
    assert b_per_tile * _NUM_TILES == n_b and b_per_tile % 128 == 0
    assert n_l % _LBAND == 0
    n_bands = n_l // _LBAND
    groups = b_per_tile // _LANES

    mesh = plsc.VectorSubcoreMesh(core_axis_name="c", subcore_axis_name="s")

    @functools.partial(
        pl.kernel,
        out_type=jax.ShapeDtypeStruct((2 * n_l, n_b), jnp.float32),
        mesh=mesh,
        scratch_types=[
            pltpu.VMEM((_LBAND, b_per_tile), jnp.int32),
            pltpu.VMEM((2 * _LBAND, b_per_tile), jnp.float32),
            pltpu.VMEM((48,), jnp.float32),
        ],
        compiler_params=pltpu.CompilerParams(needs_layout_passes=False),
    )
    def fused_lookup(ids_hbm, w_hbm, out_hbm, ids_v, out_v, wbuf):
        pltpu.sync_copy(w_hbm, wbuf)
        jj = lax.iota(jnp.int32, _LANES)
        v7 = jj & 7
        tev = plsc.load_gather(wbuf, [jnp.full((_LANES,), 40, jnp.int32)])
        tod = plsc.load_gather(wbuf, [jnp.full((_LANES,), 41, jnp.int32)])
        for d in range(4):
            e = plsc.load_gather(wbuf, [v7 * 4 + d])
            w0 = plsc.load_gather(wbuf, [jnp.full((_LANES,), 32 + d, jnp.int32)])
            w1 = plsc.load_gather(wbuf, [jnp.full((_LANES,), 36 + d, jnp.int32)])
            tev = tev + e * w0
            tod = tod + e * w1

        wid = lax.axis_index("s") * _NUM_CORES + lax.axis_index("c")
        b0 = wid * b_per_tile

        def band_body(k, carry):
            l0 = k * _LBAND
            pltpu.sync_copy(
                ids_hbm.at[pl.ds(l0, _LBAND), pl.ds(b0, b_per_tile)], ids_v
            )

            @plsc.parallel_loop(0, groups, step=1, unroll=2)
            def group_body(g):
                off = g * _LANES
                for l in range(_LBAND):
                    ids16 = ids_v[l, pl.ds(off, _LANES)]
                    out_v[2 * l, pl.ds(off, _LANES)] = jnp.take_along_axis(
                        tev, ids16, axis=0, mode="promise_in_bounds"
                    )
                    out_v[2 * l + 1, pl.ds(off, _LANES)] = jnp.take_along_axis(
                        tod, ids16, axis=0, mode="promise_in_bounds"
                    )

            pltpu.sync_copy(
                out_v, out_hbm.at[pl.ds(2 * l0, 2 * _LBAND), pl.ds(b0, b_per_tile)]
            )
            return carry

        lax.fori_loop(0, n_bands, band_body, 0)

    return fused_lookup


def kernel(input_ids, embed_weight, lm_head_weight, lm_head_bias):
    b, l = input_ids.shape
    ids_t = input_ids.astype(jnp.int32).T
    wpacked = jnp.concatenate(
        [
            embed_weight.reshape(-1).astype(jnp.float32),
            lm_head_weight.reshape(-1).astype(jnp.float32),
            lm_head_bias.astype(jnp.float32),
            jnp.zeros((6,), jnp.float32),
        ]
    )
    out2 = _build(l, b)(ids_t, wpacked)
    return out2.reshape(l, 2, b).transpose(2, 0, 1)


# speedup vs baseline: 155.8484x; 1.1496x over previous
import functools

import jax
import jax.numpy as jnp
from jax import lax
from jax.experimental import pallas as pl
from jax.experimental.pallas import tpu as pltpu
from jax.experimental.pallas import tpu_sc as plsc

_NUM_CORES = 2
_NUM_SUBCORES = 16
_NUM_TILES = _NUM_CORES * _NUM_SUBCORES
_LANES = 16

_LBAND = 8


def _build(n_l: int, n_b: int):
    b_per_tile = n_b // _NUM_TILES
    assert b_per_tile * _NUM_TILES == n_b and b_per_tile % 128 == 0
    assert n_l % _LBAND == 0
    n_bands = n_l // _LBAND
    groups = b_per_tile // _LANES

    mesh = plsc.VectorSubcoreMesh(core_axis_name="c", subcore_axis_name="s")

    @functools.partial(
        pl.kernel,
        out_type=jax.ShapeDtypeStruct((2 * n_l, n_b), jnp.float32),
        mesh=mesh,
        scratch_types=[
            pltpu.VMEM((_LBAND, b_per_tile), jnp.int32),
            pltpu.VMEM((_LBAND, b_per_tile), jnp.int32),
            pltpu.VMEM((2 * _LBAND, b_per_tile), jnp.float32),
            pltpu.VMEM((2 * _LBAND, b_per_tile), jnp.float32),
            pltpu.VMEM((48,), jnp.float32),
            pltpu.SemaphoreType.DMA,
            pltpu.SemaphoreType.DMA,
            pltpu.SemaphoreType.DMA,
            pltpu.SemaphoreType.DMA,
        ],
        compiler_params=pltpu.CompilerParams(needs_layout_passes=False),
    )
    def fused_lookup(
        ids_hbm, w_hbm, out_hbm,
        ids_v0, ids_v1, out_v0, out_v1, wbuf,
        in_sem0, in_sem1, out_sem0, out_sem1,
    ):
        ids_bufs = (ids_v0, ids_v1)
        out_bufs = (out_v0, out_v1)
        in_sems = (in_sem0, in_sem1)
        out_sems = (out_sem0, out_sem1)
        pltpu.sync_copy(w_hbm, wbuf)
        jj = lax.iota(jnp.int32, _LANES)
        v7 = jj & 7
        tev = plsc.load_gather(wbuf, [jnp.full((_LANES,), 40, jnp.int32)])
        tod = plsc.load_gather(wbuf, [jnp.full((_LANES,), 41, jnp.int32)])
        for d in range(4):
            e = plsc.load_gather(wbuf, [v7 * 4 + d])
            w0 = plsc.load_gather(wbuf, [jnp.full((_LANES,), 32 + d, jnp.int32)])
            w1 = plsc.load_gather(wbuf, [jnp.full((_LANES,), 36 + d, jnp.int32)])
            tev = tev + e * w0
            tod = tod + e * w1

        wid = lax.axis_index("s") * _NUM_CORES + lax.axis_index("c")
        b0 = wid * b_per_tile

        def start_in(k):
            return pltpu.async_copy(
                ids_hbm.at[pl.ds(k * _LBAND, _LBAND), pl.ds(b0, b_per_tile)],
                ids_bufs[k & 1],
                in_sems[k & 1],
            )

        def start_out(k):
            return pltpu.async_copy(
                out_bufs[k & 1],
                out_hbm.at[pl.ds(2 * k * _LBAND, 2 * _LBAND), pl.ds(b0, b_per_tile)],
                out_sems[k & 1],
            )

        in_handles = [None, None]
        out_handles = [None, None]
        in_handles[0] = start_in(0)
        for k in range(n_bands):
            s = k & 1
            if k + 1 < n_bands:
                in_handles[(k + 1) & 1] = start_in(k + 1)
            in_handles[s].wait()
            if out_handles[s] is not None:
                out_handles[s].wait()
            ids_v = ids_bufs[s]
            out_v = out_bufs[s]

            @plsc.parallel_loop(0, groups, step=1, unroll=2)
            def group_body(g):
                off = g * _LANES
                for l in range(_LBAND):
                    ids16 = ids_v[l, pl.ds(off, _LANES)]
                    out_v[2 * l, pl.ds(off, _LANES)] = jnp.take_along_axis(
                        tev, ids16, axis=0, mode="promise_in_bounds"
                    )
                    out_v[2 * l + 1, pl.ds(off, _LANES)] = jnp.take_along_axis(
                        tod, ids16, axis=0, mode="promise_in_bounds"
                    )

            out_handles[s] = start_out(k)
        for h in out_handles:
            if h is not None:
                h.wait()

    return fused_lookup


def kernel(input_ids, embed_weight, lm_head_weight, lm_head_bias):
    b, l = input_ids.shape
    ids_t = input_ids.astype(jnp.int32).T
    wpacked = jnp.concatenate(
        [
            embed_weight.reshape(-1).astype(jnp.float32),
            lm_head_weight.reshape(-1).astype(jnp.float32),
            lm_head_bias.astype(jnp.float32),
            jnp.zeros((6,), jnp.float32),
        ]
    )
    out2 = _build(l, b)(ids_t, wpacked)
    return out2.reshape(l, 2, b).transpose(2, 0, 1)
